# Initial kernel scaffold; baseline (speedup 1.0000x reference)
import jax
import jax.numpy as jnp
from jax import lax
from jax.experimental import pallas as pl
from jax.experimental.pallas import tpu as pltpu
from jax.experimental.pallas import tpu_sc as plsc

NUM_EMB = 1000000
D = 32
SCALE_CONST = 10.0
BATCH = 16384
HIST = 50
B_TOTAL = BATCH * HIST

NC, NS, L = 2, 16, 16
NW = NC * NS
BPW = B_TOTAL // NW
CHUNK = 1024
NCHUNK = BPW // CHUNK


def _emb_body(idx_hbm, table_hbm, out_hbm, idx_v, rows_v, sem):
    wid = lax.axis_index("s") * NC + lax.axis_index("c")
    base = wid * BPW

    @pl.loop(0, NCHUNK)
    def _chunk(g):
        off = base + g * CHUNK
        pltpu.sync_copy(idx_hbm.at[pl.ds(off, CHUNK)], idx_v)
        pltpu.async_copy(table_hbm.at[idx_v], rows_v, sem).wait()

        @pl.loop(0, CHUNK, unroll=4)
        def _row(i):
            rows_v[i, pl.ds(0, L)] = rows_v[i, pl.ds(0, L)] * SCALE_CONST
            rows_v[i, pl.ds(L, L)] = rows_v[i, pl.ds(L, L)] * SCALE_CONST

        pltpu.sync_copy(rows_v, out_hbm.at[pl.ds(off, CHUNK)])


def kernel(x, weight):
    xf = x.reshape(-1).astype(jnp.int32)
    mesh = plsc.VectorSubcoreMesh(core_axis_name="c", subcore_axis_name="s")
    out = pl.kernel(
        _emb_body,
        out_type=jax.ShapeDtypeStruct((B_TOTAL, D), jnp.float32),
        mesh=mesh,
        scratch_types=[
            pltpu.VMEM((CHUNK,), jnp.int32),
            pltpu.VMEM((CHUNK, D), jnp.float32),
            pltpu.SemaphoreType.DMA,
        ],
    )(xf, weight)
    return out.reshape(BATCH, HIST, D)


# speedup vs baseline: 1.0142x; 1.0142x over previous
import jax
import jax.numpy as jnp
from jax import lax
from jax.experimental import pallas as pl
from jax.experimental.pallas import tpu as pltpu
from jax.experimental.pallas import tpu_sc as plsc

NUM_EMB = 1000000
D = 32
SCALE_CONST = 10.0
BATCH = 16384
HIST = 50
B_TOTAL = BATCH * HIST

NC, NS, L = 2, 16, 16
NW = NC * NS
BPW = B_TOTAL // NW
CHUNK = 1024
NCHUNK = BPW // CHUNK


def _emb_body(idx_hbm, table_hbm, out_hbm, idx_v, rows_v, sem):
    wid = lax.axis_index("s") * NC + lax.axis_index("c")
    base = wid * BPW

    @pl.loop(0, NCHUNK)
    def _chunk(g):
        off = base + g * CHUNK
        pltpu.sync_copy(idx_hbm.at[pl.ds(off, CHUNK)], idx_v)
        pltpu.async_copy(table_hbm.at[idx_v], rows_v, sem).wait()

        @pl.loop(0, CHUNK, unroll=4)
        def _row(i):
            rows_v[i, pl.ds(0, L)] = rows_v[i, pl.ds(0, L)] * SCALE_CONST
            rows_v[i, pl.ds(L, L)] = rows_v[i, pl.ds(L, L)] * SCALE_CONST

        pltpu.sync_copy(rows_v, out_hbm.at[pl.ds(off, CHUNK)])


def kernel(x, weight):
    xf = x.reshape(-1).astype(jnp.int32)
    mesh = plsc.VectorSubcoreMesh(core_axis_name="c", subcore_axis_name="s")
    out = pl.kernel(
        _emb_body,
        out_type=jax.ShapeDtypeStruct((B_TOTAL, D), jnp.float32),
        mesh=mesh,
        scratch_types=[
            pltpu.VMEM((CHUNK,), jnp.int32),
            pltpu.VMEM((CHUNK, D), jnp.float32),
            pltpu.SemaphoreType.DMA,
        ],
        compiler_params=pltpu.CompilerParams(use_tc_tiling_on_sc=False),
    )(xf, weight)
    return out.reshape(BATCH, HIST, D)


# speedup vs baseline: 1.3011x; 1.2830x over previous
import jax
import jax.numpy as jnp
from jax import lax
from jax.experimental import pallas as pl
from jax.experimental.pallas import tpu as pltpu
from jax.experimental.pallas import tpu_sc as plsc

NUM_EMB = 1000000
D = 32
SCALE_CONST = 10.0
BATCH = 16384
HIST = 50
B_TOTAL = BATCH * HIST

NC, NS, L = 2, 16, 16
NW = NC * NS

EBLK = 512
NBLK = NUM_EMB // EBLK
TAIL = NUM_EMB - NBLK * EBLK
BLK_PER_W = (NBLK + NW - 1) // NW
TROW0 = (NBLK * EBLK) // 4

BPW = B_TOTAL // NW
CHUNK = 1024
NCHUNK = BPW // CHUNK


def _transpose_body(wt_hbm, wtail_hbm, table_hbm, blk_v, rows_v, tail_v):
    wid = lax.axis_index("s") * NC + lax.axis_index("c")
    iota_lo = lax.iota(jnp.int32, L)
    iota_hi = iota_lo + L

    @pl.loop(0, BLK_PER_W)
    def _blk(k):
        blk = wid + k * NW

        @pl.when(blk < NBLK)
        def _full():
            e0 = pl.multiple_of(blk * EBLK, 128)
            pltpu.sync_copy(wt_hbm.at[:, pl.ds(e0, EBLK)], blk_v)

            @pl.loop(0, EBLK, unroll=4)
            def _c(c):
                col = jnp.full((L,), c, jnp.int32)
                v0 = plsc.load_gather(blk_v, [iota_lo, col]) * SCALE_CONST
                v1 = plsc.load_gather(blk_v, [iota_hi, col]) * SCALE_CONST
                r = c >> 2
                c0 = (c & 3) * D
                rows_v[r, pl.ds(c0, L)] = v0
                rows_v[r, pl.ds(c0 + L, L)] = v1

            r0 = pl.multiple_of(blk * (EBLK // 4), 128)
            pltpu.sync_copy(rows_v, table_hbm.at[pl.ds(r0, EBLK // 4)])

    @pl.when(wid == 0)
    def _tail():
        pltpu.sync_copy(wtail_hbm, tail_v)
        pltpu.sync_copy(tail_v, table_hbm.at[pl.ds(TROW0, TAIL // 4)])


def _gather_body(xf_hbm, table_hbm, out_hbm, idx_v, rows_v, sem):
    wid = lax.axis_index("s") * NC + lax.axis_index("c")
    base = wid * BPW

    @pl.loop(0, NCHUNK)
    def _chunk(g):
        off = base + g * CHUNK
        pltpu.sync_copy(xf_hbm.at[pl.ds(off, CHUNK)], idx_v)
        pltpu.async_copy(table_hbm.at[idx_v], rows_v, sem).wait()
        pltpu.sync_copy(rows_v, out_hbm.at[pl.ds(off, CHUNK)])


def kernel(x, weight):
    wT = weight.T
    wtail = (lax.slice(weight, (NBLK * EBLK, 0), (NUM_EMB, D))
             * SCALE_CONST).reshape(TAIL // 4, 128)
    xf = x.astype(jnp.int32).T.reshape(B_TOTAL)
    mesh = plsc.VectorSubcoreMesh(core_axis_name="c", subcore_axis_name="s")

    table128 = pl.kernel(
        _transpose_body,
        out_type=jax.ShapeDtypeStruct((NUM_EMB // 4, 128), jnp.float32),
        mesh=mesh,
        scratch_types=[
            pltpu.VMEM((D, EBLK), jnp.float32),
            pltpu.VMEM((EBLK // 4, 128), jnp.float32),
            pltpu.VMEM((TAIL // 4, 128), jnp.float32),
        ],
        compiler_params=pltpu.CompilerParams(needs_layout_passes=False),
    )(wT, wtail)

    table = table128.reshape(NUM_EMB, D)

    out2 = pl.kernel(
        _gather_body,
        out_type=jax.ShapeDtypeStruct((B_TOTAL, D), jnp.float32),
        mesh=mesh,
        scratch_types=[
            pltpu.VMEM((CHUNK,), jnp.int32),
            pltpu.VMEM((CHUNK, D), jnp.float32),
            pltpu.SemaphoreType.DMA,
        ],
        compiler_params=pltpu.CompilerParams(use_tc_tiling_on_sc=False),
    )(xf, table)

    return out2.reshape(HIST, BATCH, D).transpose(1, 0, 2)


# speedup vs baseline: 1.4374x; 1.1047x over previous
import jax
import jax.numpy as jnp
from jax import lax
from jax.experimental import pallas as pl
from jax.experimental.pallas import tpu as pltpu
from jax.experimental.pallas import tpu_sc as plsc

NUM_EMB = 1000000
D = 32
SCALE_CONST = 10.0
BATCH = 16384
HIST = 50
B_TOTAL = BATCH * HIST

NC, NS, L = 2, 16, 16
NW = NC * NS

EBLK = 512
NBLK = NUM_EMB // EBLK
TAIL = NUM_EMB - NBLK * EBLK
BLK_PER_W = (NBLK + NW - 1) // NW
TROW0 = (NBLK * EBLK) // 4

BPW = B_TOTAL // NW
CHUNK = 1024
NCHUNK = BPW // CHUNK


def _transpose_body(wt_hbm, wtail_hbm, table_hbm,
                    blk0_v, blk1_v, rows0_v, rows1_v, tail_v,
                    sin0, sin1, sout0, sout1):
    wid = lax.axis_index("s") * NC + lax.axis_index("c")
    iota_lo = lax.iota(jnp.int32, L)
    iota_hi = iota_lo + L
    blks = (blk0_v, blk1_v)
    rows = (rows0_v, rows1_v)
    sins = (sin0, sin1)
    souts = (sout0, sout1)
    nb = jnp.where(wid == 0, BLK_PER_W, BLK_PER_W - 1)

    def _src(k):
        e0 = pl.multiple_of((wid + k * NW) * EBLK, 128)
        return wt_hbm.at[:, pl.ds(e0, EBLK)]

    def _dst(k):
        r0 = pl.multiple_of((wid + k * NW) * (EBLK // 4), 128)
        return table_hbm.at[pl.ds(r0, EBLK // 4)]

    pltpu.async_copy(_src(0), blks[0], sins[0])

    @pl.loop(0, BLK_PER_W // 2)
    def _j(j):
        for p in range(2):
            k = j * 2 + p

            @pl.when(k < nb)
            def _():
                pltpu.make_async_copy(_src(k), blks[p], sins[p]).wait()

                @pl.when(k + 1 < nb)
                def _():
                    pltpu.async_copy(_src(k + 1), blks[1 - p], sins[1 - p])

                @pl.when(k >= 2)
                def _():
                    pltpu.make_async_copy(rows[p], _dst(k - 2), souts[p]).wait()

                @pl.loop(0, EBLK // 4, unroll=2)
                def _r(r):
                    c4 = r * 4
                    for q in range(4):
                        col = jnp.full((L,), c4 + q, jnp.int32)
                        v0 = plsc.load_gather(blks[p], [iota_lo, col])
                        v1 = plsc.load_gather(blks[p], [iota_hi, col])
                        rows[p][r, pl.ds(q * D, L)] = v0 * SCALE_CONST
                        rows[p][r, pl.ds(q * D + L, L)] = v1 * SCALE_CONST

                pltpu.async_copy(rows[p], _dst(k), souts[p])

    pltpu.make_async_copy(rows0_v, _dst(nb - 2 + (nb & 1)), sout0).wait()
    pltpu.make_async_copy(rows1_v, _dst(nb - 1 - (nb & 1)), sout1).wait()

    @pl.when(wid == 0)
    def _tail():
        pltpu.sync_copy(wtail_hbm, tail_v)
        pltpu.sync_copy(tail_v, table_hbm.at[pl.ds(TROW0, TAIL // 4)])


def _gather_body(xf_hbm, table_hbm, out_hbm, idx_v, rows_v, sem):
    wid = lax.axis_index("s") * NC + lax.axis_index("c")
    base = wid * BPW

    @pl.loop(0, NCHUNK)
    def _chunk(g):
        off = base + g * CHUNK
        pltpu.sync_copy(xf_hbm.at[pl.ds(off, CHUNK)], idx_v)
        pltpu.async_copy(table_hbm.at[idx_v], rows_v, sem).wait()
        pltpu.sync_copy(rows_v, out_hbm.at[pl.ds(off, CHUNK)])


def kernel(x, weight):
    wT = weight.T
    wtail = (lax.slice(weight, (NBLK * EBLK, 0), (NUM_EMB, D))
             * SCALE_CONST).reshape(TAIL // 4, 128)
    xf = x.astype(jnp.int32).T.reshape(B_TOTAL)
    mesh = plsc.VectorSubcoreMesh(core_axis_name="c", subcore_axis_name="s")

    table128 = pl.kernel(
        _transpose_body,
        out_type=jax.ShapeDtypeStruct((NUM_EMB // 4, 128), jnp.float32),
        mesh=mesh,
        scratch_types=[
            pltpu.VMEM((D, EBLK), jnp.float32),
            pltpu.VMEM((D, EBLK), jnp.float32),
            pltpu.VMEM((EBLK // 4, 128), jnp.float32),
            pltpu.VMEM((EBLK // 4, 128), jnp.float32),
            pltpu.VMEM((TAIL // 4, 128), jnp.float32),
            pltpu.SemaphoreType.DMA,
            pltpu.SemaphoreType.DMA,
            pltpu.SemaphoreType.DMA,
            pltpu.SemaphoreType.DMA,
        ],
        compiler_params=pltpu.CompilerParams(needs_layout_passes=False),
    )(wT, wtail)

    table = table128.reshape(NUM_EMB, D)

    out2 = pl.kernel(
        _gather_body,
        out_type=jax.ShapeDtypeStruct((B_TOTAL, D), jnp.float32),
        mesh=mesh,
        scratch_types=[
            pltpu.VMEM((CHUNK,), jnp.int32),
            pltpu.VMEM((CHUNK, D), jnp.float32),
            pltpu.SemaphoreType.DMA,
        ],
        compiler_params=pltpu.CompilerParams(use_tc_tiling_on_sc=False),
    )(xf, table)

    return out2.reshape(HIST, BATCH, D).transpose(1, 0, 2)
